# baseline (speedup 1.0000x reference)
import functools

import jax
import jax.numpy as jnp
from jax import lax
from jax.experimental import pallas as pl
from jax.experimental.pallas import tpu as pltpu
from jax.experimental.pallas import tpu_sc as plsc

N = 10000
E = 320000
D = 128
G = 64
OUT = 96
C = 5
EPS = 1e-5

NC = 2
NS = 16
DH = D // NC
LANES = 128
CH = 162
EPW = CH * LANES
E_PAD = NS * EPW
N_PAD = 10112
ROWS_PT = N_PAD // NS
K = 3
NG = CH // K

RBLK = 1000
NBLK = N // RBLK

_HI = lax.Precision.HIGHEST


def _sc_scatter_add(x2, src_t, dst_t, zrows):
  mesh = plsc.VectorSubcoreMesh(
      core_axis_name="c", subcore_axis_name="s", num_cores=NC, num_subcores=NS)

  @functools.partial(
      pl.kernel,
      mesh=mesh,
      out_type=jax.ShapeDtypeStruct((NC, N_PAD, DH), jnp.float32),
      scratch_types=[
          pltpu.VMEM((CH, LANES), jnp.int32),
          pltpu.VMEM((CH, LANES), jnp.int32),
          pltpu.VMEM((K * LANES, DH), jnp.float32),
          pltpu.VMEM((K * LANES, DH), jnp.float32),
          pltpu.SemaphoreType.DMA,
          pltpu.SemaphoreType.DMA,
          pltpu.VMEM_SHARED((N_PAD, DH), jnp.float32),
      ],
      compiler_params=pltpu.CompilerParams(use_tc_tiling_on_sc=False),
  )
  def k(x_hbm, src_hbm, dst_hbm, z_hbm, out_hbm,
        src_v, dst_v, buf_a, buf_b, gsem, ssem, agg_sh):
    cid = lax.axis_index("c")
    sid = lax.axis_index("s")
    xc = x_hbm.at[cid]
    pltpu.sync_copy(z_hbm, agg_sh.at[pl.ds(sid * ROWS_PT, ROWS_PT)])
    pltpu.sync_copy(src_hbm.at[sid], src_v)
    pltpu.sync_copy(dst_hbm.at[sid], dst_v)
    plsc.subcore_barrier()

    def fire_gathers(g, buf, sem):
      for j in range(K):
        pltpu.async_copy(xc.at[src_v.at[K * g + j]],
                         buf.at[pl.ds(j * LANES, LANES)], sem)

    def drain(buf, sem, n=K):
      for j in range(n):
        pltpu.make_async_copy(xc.at[pl.ds(0, LANES)],
                              buf.at[pl.ds(j * LANES, LANES)], sem).wait()

    def fire_scatters(g, buf, sem):
      for j in range(K):
        pltpu.async_copy(buf.at[pl.ds(j * LANES, LANES)],
                         agg_sh.at[dst_v.at[K * g + j]], sem, add=True)

    fire_gathers(0, buf_a, gsem)

    def body(g, carry):
      def step(buf, obuf):
        drain(buf, gsem)

        @pl.when(g + 1 < NG)
        def _():
          @pl.when(g >= 1)
          def _():
            drain(obuf, ssem)
          fire_gathers(g + 1, obuf, gsem)

        fire_scatters(g, buf, ssem)

      @pl.when(g % 2 == 0)
      def _():
        step(buf_a, buf_b)

      @pl.when(g % 2 == 1)
      def _():
        step(buf_b, buf_a)

      return carry

    lax.fori_loop(0, NG, body, 0)
    if NG % 2 == 0:
      drain(buf_a, ssem)
      drain(buf_b, ssem)
    else:
      drain(buf_b, ssem)
      drain(buf_a, ssem)
    plsc.subcore_barrier()
    pltpu.sync_copy(agg_sh.at[pl.ds(sid * ROWS_PT, ROWS_PT)],
                    out_hbm.at[cid, pl.ds(sid * ROWS_PT, ROWS_PT)])

  return k(x2, src_t, dst_t, zrows)


def _tail_body(agg_ref, x_ref, b3_ref, wrel_ref, wroot_ref, bc_ref,
               bnw_ref, bnb_ref, cw_ref, cb_ref, o_ref,
               wr_s, wt_s, bs_s, cs_s, cq_s, gmax_s, gmin_s):
  i = pl.program_id(0)

  @pl.when(i == 0)
  def _init():
    wr_s[...] = jnp.sum(wrel_ref[...], axis=0)
    wt_s[...] = jnp.sum(wroot_ref[...], axis=0)
    bs_s[...] = jnp.sum(bc_ref[...], axis=0, keepdims=True)
    cs_s[...] = jnp.zeros_like(cs_s)
    cq_s[...] = jnp.zeros_like(cq_s)
    gmax_s[...] = jnp.full_like(gmax_s, -jnp.inf)
    gmin_s[...] = jnp.full_like(gmin_s, jnp.inf)

  wr = wr_s[...]
  out = (jnp.dot(agg_ref[0], wr[:DH, :], preferred_element_type=jnp.float32,
                 precision=_HI)
         + jnp.dot(agg_ref[1], wr[DH:, :], preferred_element_type=jnp.float32,
                   precision=_HI)
         + jnp.dot(x_ref[...], wt_s[...], preferred_element_type=jnp.float32,
                   precision=_HI)
         + bs_s[...])
  cs_s[...] += jnp.sum(out, axis=0, keepdims=True)
  cq_s[...] += jnp.sum(out * out, axis=0, keepdims=True)

  b = b3_ref[0]
  g_lo = jnp.min(b)
  g_hi = jnp.max(b)

  def upd(g, carry):
    m = b == g
    mx = jnp.max(jnp.where(m, out, -jnp.inf), axis=0, keepdims=True)
    mn = jnp.min(jnp.where(m, out, jnp.inf), axis=0, keepdims=True)
    row = lax.broadcasted_iota(jnp.int32, (G, 1), 0) == g
    gmax_s[...] = jnp.where(row, jnp.maximum(gmax_s[...], mx), gmax_s[...])
    gmin_s[...] = jnp.where(row, jnp.minimum(gmin_s[...], mn), gmin_s[...])
    return carry

  lax.fori_loop(g_lo, g_hi + 1, upd, 0)

  @pl.when(i == NBLK - 1)
  def _fin():
    mean = cs_s[...] / N
    var = jnp.maximum(cq_s[...] / N - mean * mean, 0.0)
    a = bnw_ref[...] * lax.rsqrt(var + EPS)
    sh = bnb_ref[...] - mean * a
    gmax = gmax_s[...]
    gmin = gmin_s[...]
    hg = jnp.where(a >= 0.0, gmax * a + sh, gmin * a + sh)
    hg = jnp.where(gmax == -jnp.inf, -jnp.inf, hg)
    gpool = jnp.maximum(hg, 0.0)
    o_ref[...] = (jnp.dot(gpool, cw_ref[...],
                          preferred_element_type=jnp.float32, precision=_HI)
                  + cb_ref[...])


def _tc_tail(aggp, x, batch3, W_rel, W_root, b_conv, bnw2, bnb2, cls_W, cls_b2):
  return pl.pallas_call(
      _tail_body,
      grid=(NBLK,),
      in_specs=[
          pl.BlockSpec((NC, RBLK, DH), lambda i: (0, i, 0)),
          pl.BlockSpec((RBLK, D), lambda i: (i, 0)),
          pl.BlockSpec((1, RBLK, 1), lambda i: (i, 0, 0)),
          pl.BlockSpec((C, D, D), lambda i: (0, 0, 0)),
          pl.BlockSpec((C, D, D), lambda i: (0, 0, 0)),
          pl.BlockSpec((C, D), lambda i: (0, 0)),
          pl.BlockSpec((1, D), lambda i: (0, 0)),
          pl.BlockSpec((1, D), lambda i: (0, 0)),
          pl.BlockSpec((D, OUT), lambda i: (0, 0)),
          pl.BlockSpec((1, OUT), lambda i: (0, 0)),
      ],
      out_specs=pl.BlockSpec((G, OUT), lambda i: (0, 0)),
      out_shape=jax.ShapeDtypeStruct((G, OUT), jnp.float32),
      scratch_shapes=[
          pltpu.VMEM((D, D), jnp.float32),
          pltpu.VMEM((D, D), jnp.float32),
          pltpu.VMEM((1, D), jnp.float32),
          pltpu.VMEM((1, D), jnp.float32),
          pltpu.VMEM((1, D), jnp.float32),
          pltpu.VMEM((G, D), jnp.float32),
          pltpu.VMEM((G, D), jnp.float32),
      ],
      compiler_params=pltpu.CompilerParams(
          dimension_semantics=("arbitrary",)),
  )(aggp, x, batch3, W_rel, W_root, b_conv, bnw2, bnb2, cls_W, cls_b2)


@jax.jit
def kernel(x, edge_index, batch, i, W_rel, W_root, b_conv, bn_w, bn_b,
           cls_W, cls_b):
  del i
  pad = E_PAD - E
  src_t = jnp.concatenate(
      [edge_index[0], jnp.zeros((pad,), jnp.int32)]).reshape(NS, CH, LANES)
  dst_t = jnp.concatenate(
      [edge_index[1], jnp.full((pad,), N, jnp.int32)]).reshape(NS, CH, LANES)
  zrows = jnp.zeros((ROWS_PT, DH), jnp.float32)
  x2 = jnp.stack([x[:, :DH], x[:, DH:]])
  aggp = _sc_scatter_add(x2, src_t, dst_t, zrows)
  return _tc_tail(aggp, x, batch.reshape(NBLK, RBLK, 1),
                  W_rel, W_root, b_conv, bn_w.reshape(1, D),
                  bn_b.reshape(1, D), cls_W, cls_b.reshape(1, OUT))


# speedup vs baseline: 1.2142x; 1.2142x over previous
import functools

import jax
import jax.numpy as jnp
from jax import lax
from jax.experimental import pallas as pl
from jax.experimental.pallas import tpu as pltpu
from jax.experimental.pallas import tpu_sc as plsc

N = 10000
E = 320000
D = 128
G = 64
OUT = 96
C = 5
EPS = 1e-5

NC = 2
NS = 16
DH = D // NC
LANES = 128
CH = 160
EPW = CH * LANES
E_PAD = NS * EPW
N_PAD = 10112
ROWS_PT = N_PAD // NS
K = 2
NG = CH // K

RBLK = 1000
NBLK = N // RBLK

_HI = lax.Precision.HIGHEST


def _sc_scatter_add(x2, src_t, dst_t, zrows):
  mesh = plsc.VectorSubcoreMesh(
      core_axis_name="c", subcore_axis_name="s", num_cores=NC, num_subcores=NS)

  @functools.partial(
      pl.kernel,
      mesh=mesh,
      out_type=jax.ShapeDtypeStruct((NC, N_PAD, DH), jnp.float32),
      scratch_types=[
          pltpu.VMEM((CH, LANES), jnp.int32),
          pltpu.VMEM((CH, LANES), jnp.int32),
          pltpu.VMEM((K * LANES, DH), jnp.float32),
          pltpu.VMEM((K * LANES, DH), jnp.float32),
          pltpu.SemaphoreType.DMA,
          pltpu.SemaphoreType.DMA,
          pltpu.VMEM_SHARED((N_PAD, DH), jnp.float32),
      ],
      compiler_params=pltpu.CompilerParams(use_tc_tiling_on_sc=False),
  )
  def k(x_hbm, src_hbm, dst_hbm, z_hbm, out_hbm,
        src_v, dst_v, buf_a, buf_b, gsem, ssem, agg_sh):
    cid = lax.axis_index("c")
    sid = lax.axis_index("s")
    xc = x_hbm.at[cid]
    pltpu.sync_copy(z_hbm, agg_sh.at[pl.ds(sid * ROWS_PT, ROWS_PT)])
    pltpu.sync_copy(src_hbm.at[sid], src_v)
    pltpu.sync_copy(dst_hbm.at[sid], dst_v)
    plsc.subcore_barrier()

    def fire_gathers(g, buf, sem):
      for j in range(K):
        pltpu.async_copy(xc.at[src_v.at[K * g + j]],
                         buf.at[pl.ds(j * LANES, LANES)], sem)

    def drain(buf, sem, n=K):
      for j in range(n):
        pltpu.make_async_copy(xc.at[pl.ds(0, LANES)],
                              buf.at[pl.ds(j * LANES, LANES)], sem).wait()

    def fire_scatters(g, buf, sem):
      for j in range(K):
        pltpu.async_copy(buf.at[pl.ds(j * LANES, LANES)],
                         agg_sh.at[dst_v.at[K * g + j]], sem, add=True)

    fire_gathers(0, buf_a, gsem)

    def body(g, carry):
      def step(buf, obuf):
        drain(buf, gsem)

        @pl.when(g + 1 < NG)
        def _():
          @pl.when(g >= 1)
          def _():
            drain(obuf, ssem)
          fire_gathers(g + 1, obuf, gsem)

        fire_scatters(g, buf, ssem)

      @pl.when(g % 2 == 0)
      def _():
        step(buf_a, buf_b)

      @pl.when(g % 2 == 1)
      def _():
        step(buf_b, buf_a)

      return carry

    lax.fori_loop(0, NG, body, 0)
    if NG % 2 == 0:
      drain(buf_a, ssem)
      drain(buf_b, ssem)
    else:
      drain(buf_b, ssem)
      drain(buf_a, ssem)
    plsc.subcore_barrier()
    pltpu.sync_copy(agg_sh.at[pl.ds(sid * ROWS_PT, ROWS_PT)],
                    out_hbm.at[cid, pl.ds(sid * ROWS_PT, ROWS_PT)])

  return k(x2, src_t, dst_t, zrows)


def _tail_body(agg_ref, x_ref, b3_ref, wrel_ref, wroot_ref, bc_ref,
               bnw_ref, bnb_ref, cw_ref, cb_ref, o_ref,
               wr_s, wt_s, bs_s, cs_s, cq_s, gmax_s, gmin_s):
  i = pl.program_id(0)

  @pl.when(i == 0)
  def _init():
    wr_s[...] = jnp.sum(wrel_ref[...], axis=0)
    wt_s[...] = jnp.sum(wroot_ref[...], axis=0)
    bs_s[...] = jnp.sum(bc_ref[...], axis=0, keepdims=True)
    cs_s[...] = jnp.zeros_like(cs_s)
    cq_s[...] = jnp.zeros_like(cq_s)
    gmax_s[...] = jnp.full_like(gmax_s, -jnp.inf)
    gmin_s[...] = jnp.full_like(gmin_s, jnp.inf)

  wr = wr_s[...]
  out = (jnp.dot(agg_ref[0], wr[:DH, :], preferred_element_type=jnp.float32,
                 precision=_HI)
         + jnp.dot(agg_ref[1], wr[DH:, :], preferred_element_type=jnp.float32,
                   precision=_HI)
         + jnp.dot(x_ref[...], wt_s[...], preferred_element_type=jnp.float32,
                   precision=_HI)
         + bs_s[...])
  cs_s[...] += jnp.sum(out, axis=0, keepdims=True)
  cq_s[...] += jnp.sum(out * out, axis=0, keepdims=True)

  b = b3_ref[0]
  g_lo = jnp.min(b)
  g_hi = jnp.max(b)

  def upd(g, carry):
    m = b == g
    mx = jnp.max(jnp.where(m, out, -jnp.inf), axis=0, keepdims=True)
    mn = jnp.min(jnp.where(m, out, jnp.inf), axis=0, keepdims=True)
    row = lax.broadcasted_iota(jnp.int32, (G, 1), 0) == g
    gmax_s[...] = jnp.where(row, jnp.maximum(gmax_s[...], mx), gmax_s[...])
    gmin_s[...] = jnp.where(row, jnp.minimum(gmin_s[...], mn), gmin_s[...])
    return carry

  lax.fori_loop(g_lo, g_hi + 1, upd, 0)

  @pl.when(i == NBLK - 1)
  def _fin():
    mean = cs_s[...] / N
    var = jnp.maximum(cq_s[...] / N - mean * mean, 0.0)
    a = bnw_ref[...] * lax.rsqrt(var + EPS)
    sh = bnb_ref[...] - mean * a
    gmax = gmax_s[...]
    gmin = gmin_s[...]
    hg = jnp.where(a >= 0.0, gmax * a + sh, gmin * a + sh)
    hg = jnp.where(gmax == -jnp.inf, -jnp.inf, hg)
    gpool = jnp.maximum(hg, 0.0)
    o_ref[...] = (jnp.dot(gpool, cw_ref[...],
                          preferred_element_type=jnp.float32, precision=_HI)
                  + cb_ref[...])


def _tc_tail(aggp, x, batch3, W_rel, W_root, b_conv, bnw2, bnb2, cls_W, cls_b2):
  return pl.pallas_call(
      _tail_body,
      grid=(NBLK,),
      in_specs=[
          pl.BlockSpec((NC, RBLK, DH), lambda i: (0, i, 0)),
          pl.BlockSpec((RBLK, D), lambda i: (i, 0)),
          pl.BlockSpec((1, RBLK, 1), lambda i: (i, 0, 0)),
          pl.BlockSpec((C, D, D), lambda i: (0, 0, 0)),
          pl.BlockSpec((C, D, D), lambda i: (0, 0, 0)),
          pl.BlockSpec((C, D), lambda i: (0, 0)),
          pl.BlockSpec((1, D), lambda i: (0, 0)),
          pl.BlockSpec((1, D), lambda i: (0, 0)),
          pl.BlockSpec((D, OUT), lambda i: (0, 0)),
          pl.BlockSpec((1, OUT), lambda i: (0, 0)),
      ],
      out_specs=pl.BlockSpec((G, OUT), lambda i: (0, 0)),
      out_shape=jax.ShapeDtypeStruct((G, OUT), jnp.float32),
      scratch_shapes=[
          pltpu.VMEM((D, D), jnp.float32),
          pltpu.VMEM((D, D), jnp.float32),
          pltpu.VMEM((1, D), jnp.float32),
          pltpu.VMEM((1, D), jnp.float32),
          pltpu.VMEM((1, D), jnp.float32),
          pltpu.VMEM((G, D), jnp.float32),
          pltpu.VMEM((G, D), jnp.float32),
      ],
      compiler_params=pltpu.CompilerParams(
          dimension_semantics=("arbitrary",)),
  )(aggp, x, batch3, W_rel, W_root, b_conv, bnw2, bnb2, cls_W, cls_b2)


@jax.jit
def kernel(x, edge_index, batch, i, W_rel, W_root, b_conv, bn_w, bn_b,
           cls_W, cls_b):
  del i
  pad = E_PAD - E
  src_t = jnp.concatenate(
      [edge_index[0], jnp.zeros((pad,), jnp.int32)]).reshape(NS, CH, LANES)
  dst_t = jnp.concatenate(
      [edge_index[1], jnp.full((pad,), N, jnp.int32)]).reshape(NS, CH, LANES)
  zrows = jnp.zeros((ROWS_PT, DH), jnp.float32)
  x2 = jnp.stack([x[:, :DH], x[:, DH:]])
  aggp = _sc_scatter_add(x2, src_t, dst_t, zrows)
  return _tc_tail(aggp, x, batch.reshape(NBLK, RBLK, 1),
                  W_rel, W_root, b_conv, bn_w.reshape(1, D),
                  bn_b.reshape(1, D), cls_W, cls_b.reshape(1, OUT))


# speedup vs baseline: 1.2373x; 1.0190x over previous
import functools

import jax
import jax.numpy as jnp
from jax import lax
from jax.experimental import pallas as pl
from jax.experimental.pallas import tpu as pltpu
from jax.experimental.pallas import tpu_sc as plsc

N = 10000
E = 320000
D = 128
G = 64
OUT = 96
C = 5
EPS = 1e-5

NC = 2
NS = 16
DH = D // NC
LANES = 128
CH = 160
EPW = CH * LANES
E_PAD = NS * EPW
N_PAD = 10112
ROWS_PT = N_PAD // NS
K = 2
NG = CH // K

RBLK = 1000
NBLK = N // RBLK

_HI = lax.Precision.HIGHEST


def _sc_scatter_add(x2, src_t, dst_t, zrows):
  mesh = plsc.VectorSubcoreMesh(
      core_axis_name="c", subcore_axis_name="s", num_cores=NC, num_subcores=NS)

  @functools.partial(
      pl.kernel,
      mesh=mesh,
      out_type=jax.ShapeDtypeStruct((NC, N_PAD, DH), jnp.float32),
      scratch_types=[
          pltpu.VMEM((CH, LANES), jnp.int32),
          pltpu.VMEM((CH, LANES), jnp.int32),
          pltpu.VMEM((K * LANES, DH), jnp.float32),
          pltpu.VMEM((K * LANES, DH), jnp.float32),
          pltpu.SemaphoreType.DMA,
          pltpu.SemaphoreType.DMA,
          pltpu.VMEM_SHARED((N_PAD, DH), jnp.float32),
      ],
      compiler_params=pltpu.CompilerParams(use_tc_tiling_on_sc=False),
  )
  def k(x_hbm, src_hbm, dst_hbm, z_hbm, out_hbm,
        src_v, dst_v, buf_a, buf_b, gsem, ssem, agg_sh):
    cid = lax.axis_index("c")
    sid = lax.axis_index("s")
    xc = x_hbm.at[cid]
    pltpu.sync_copy(z_hbm, agg_sh.at[pl.ds(sid * ROWS_PT, ROWS_PT)])
    pltpu.sync_copy(src_hbm.at[sid], src_v)
    pltpu.sync_copy(dst_hbm.at[sid], dst_v)
    plsc.subcore_barrier()

    def fire_gathers(g, buf, sem):
      for j in range(K):
        pltpu.async_copy(xc.at[src_v.at[K * g + j]],
                         buf.at[pl.ds(j * LANES, LANES)], sem)

    def drain(buf, sem, n=K):
      for j in range(n):
        pltpu.make_async_copy(xc.at[pl.ds(0, LANES)],
                              buf.at[pl.ds(j * LANES, LANES)], sem).wait()

    def fire_scatters(g, buf, sem):
      for j in range(K):
        pltpu.async_copy(buf.at[pl.ds(j * LANES, LANES)],
                         agg_sh.at[dst_v.at[K * g + j]], sem, add=True)

    fire_gathers(0, buf_a, gsem)

    def body(g, carry):
      def step(buf, obuf):
        drain(buf, gsem)

        @pl.when(g + 1 < NG)
        def _():
          @pl.when(g >= 1)
          def _():
            drain(obuf, ssem)
          fire_gathers(g + 1, obuf, gsem)

        fire_scatters(g, buf, ssem)

      @pl.when(g % 2 == 0)
      def _():
        step(buf_a, buf_b)

      @pl.when(g % 2 == 1)
      def _():
        step(buf_b, buf_a)

      return carry

    lax.fori_loop(0, NG, body, 0)
    if NG % 2 == 0:
      drain(buf_a, ssem)
      drain(buf_b, ssem)
    else:
      drain(buf_b, ssem)
      drain(buf_a, ssem)
    plsc.subcore_barrier()
    pltpu.sync_copy(agg_sh.at[pl.ds(sid * ROWS_PT, ROWS_PT)],
                    out_hbm.at[cid, pl.ds(sid * ROWS_PT, ROWS_PT)])

  return k(x2, src_t, dst_t, zrows)


def _front_body(x_ref, wroot_ref, bc_ref, o_ref, wt_s, bs_s):
  i = pl.program_id(0)

  @pl.when(i == 0)
  def _init():
    wt_s[...] = jnp.sum(wroot_ref[...], axis=0)
    bs_s[...] = jnp.sum(bc_ref[...], axis=0, keepdims=True)

  o_ref[...] = (jnp.dot(x_ref[...], wt_s[...],
                        preferred_element_type=jnp.float32, precision=_HI)
                + bs_s[...])


def _tc_front(x, W_root, b_conv):
  return pl.pallas_call(
      _front_body,
      grid=(NBLK,),
      in_specs=[
          pl.BlockSpec((RBLK, D), lambda i: (i, 0)),
          pl.BlockSpec((C, D, D), lambda i: (0, 0, 0)),
          pl.BlockSpec((C, D), lambda i: (0, 0)),
      ],
      out_specs=pl.BlockSpec((RBLK, D), lambda i: (i, 0)),
      out_shape=jax.ShapeDtypeStruct((N, D), jnp.float32),
      scratch_shapes=[
          pltpu.VMEM((D, D), jnp.float32),
          pltpu.VMEM((1, D), jnp.float32),
      ],
      compiler_params=pltpu.CompilerParams(
          dimension_semantics=("arbitrary",)),
  )(x, W_root, b_conv)


def _tail_body(agg_ref, xr_ref, b3_ref, wrel_ref,
               bnw_ref, bnb_ref, cw_ref, cb_ref, o_ref,
               wr_s, cs_s, cq_s, gmax_s, gmin_s):
  i = pl.program_id(0)

  @pl.when(i == 0)
  def _init():
    wr_s[...] = jnp.sum(wrel_ref[...], axis=0)
    cs_s[...] = jnp.zeros_like(cs_s)
    cq_s[...] = jnp.zeros_like(cq_s)
    gmax_s[...] = jnp.full_like(gmax_s, -jnp.inf)
    gmin_s[...] = jnp.full_like(gmin_s, jnp.inf)

  wr = wr_s[...]
  out = (jnp.dot(agg_ref[0], wr[:DH, :], preferred_element_type=jnp.float32,
                 precision=_HI)
         + jnp.dot(agg_ref[1], wr[DH:, :], preferred_element_type=jnp.float32,
                   precision=_HI)
         + xr_ref[...])
  cs_s[...] += jnp.sum(out, axis=0, keepdims=True)
  cq_s[...] += jnp.sum(out * out, axis=0, keepdims=True)

  b = b3_ref[0]
  g_lo = jnp.min(b)
  g_hi = jnp.max(b)

  def upd(g, carry):
    m = b == g
    mx = jnp.max(jnp.where(m, out, -jnp.inf), axis=0, keepdims=True)
    mn = jnp.min(jnp.where(m, out, jnp.inf), axis=0, keepdims=True)
    row = lax.broadcasted_iota(jnp.int32, (G, 1), 0) == g
    gmax_s[...] = jnp.where(row, jnp.maximum(gmax_s[...], mx), gmax_s[...])
    gmin_s[...] = jnp.where(row, jnp.minimum(gmin_s[...], mn), gmin_s[...])
    return carry

  lax.fori_loop(g_lo, g_hi + 1, upd, 0)

  @pl.when(i == NBLK - 1)
  def _fin():
    mean = cs_s[...] / N
    var = jnp.maximum(cq_s[...] / N - mean * mean, 0.0)
    a = bnw_ref[...] * lax.rsqrt(var + EPS)
    sh = bnb_ref[...] - mean * a
    gmax = gmax_s[...]
    gmin = gmin_s[...]
    hg = jnp.where(a >= 0.0, gmax * a + sh, gmin * a + sh)
    hg = jnp.where(gmax == -jnp.inf, -jnp.inf, hg)
    gpool = jnp.maximum(hg, 0.0)
    o_ref[...] = (jnp.dot(gpool, cw_ref[...],
                          preferred_element_type=jnp.float32, precision=_HI)
                  + cb_ref[...])


def _tc_tail(aggp, xr, batch3, W_rel, bnw2, bnb2, cls_W, cls_b2):
  return pl.pallas_call(
      _tail_body,
      grid=(NBLK,),
      in_specs=[
          pl.BlockSpec((NC, RBLK, DH), lambda i: (0, i, 0)),
          pl.BlockSpec((RBLK, D), lambda i: (i, 0)),
          pl.BlockSpec((1, RBLK, 1), lambda i: (i, 0, 0)),
          pl.BlockSpec((C, D, D), lambda i: (0, 0, 0)),
          pl.BlockSpec((1, D), lambda i: (0, 0)),
          pl.BlockSpec((1, D), lambda i: (0, 0)),
          pl.BlockSpec((D, OUT), lambda i: (0, 0)),
          pl.BlockSpec((1, OUT), lambda i: (0, 0)),
      ],
      out_specs=pl.BlockSpec((G, OUT), lambda i: (0, 0)),
      out_shape=jax.ShapeDtypeStruct((G, OUT), jnp.float32),
      scratch_shapes=[
          pltpu.VMEM((D, D), jnp.float32),
          pltpu.VMEM((1, D), jnp.float32),
          pltpu.VMEM((1, D), jnp.float32),
          pltpu.VMEM((G, D), jnp.float32),
          pltpu.VMEM((G, D), jnp.float32),
      ],
      compiler_params=pltpu.CompilerParams(
          dimension_semantics=("arbitrary",)),
  )(aggp, xr, batch3, W_rel, bnw2, bnb2, cls_W, cls_b2)


@jax.jit
def kernel(x, edge_index, batch, i, W_rel, W_root, b_conv, bn_w, bn_b,
           cls_W, cls_b):
  del i
  pad = E_PAD - E
  src_t = jnp.concatenate(
      [edge_index[0], jnp.zeros((pad,), jnp.int32)]).reshape(NS, CH, LANES)
  dst_t = jnp.concatenate(
      [edge_index[1], jnp.full((pad,), N, jnp.int32)]).reshape(NS, CH, LANES)
  zrows = jnp.zeros((ROWS_PT, DH), jnp.float32)
  x2 = jnp.stack([x[:, :DH], x[:, DH:]])
  aggp = _sc_scatter_add(x2, src_t, dst_t, zrows)
  xr = _tc_front(x, W_root, b_conv)
  return _tc_tail(aggp, xr, batch.reshape(NBLK, RBLK, 1),
                  W_rel, bn_w.reshape(1, D),
                  bn_b.reshape(1, D), cls_W, cls_b.reshape(1, OUT))
